# initial kernel scaffold (unmeasured)
import jax
import jax.numpy as jnp
from jax import lax
from jax.experimental import pallas as pl
from jax.experimental.pallas import tpu as pltpu

N_Y = 4
T = 512
TPD = T // N_Y
D = 512
E = 8
EPD = E // N_Y
F = 1024


def _body(x_ref, rt_ref, w1_ref, w2_ref, out_ref,
          xg_ref, rg_ref, contrib_ref, rs_ref,
          ag_ss, ag_rs, rg_ss, rg_rs, rs_ss, rs_rs):
    my_x = lax.axis_index("x")
    my_y = lax.axis_index("y")
    my_z = lax.axis_index("z")
    left = (my_y - 1) % N_Y
    right = (my_y + 1) % N_Y

    barrier = pltpu.get_barrier_semaphore()
    for nbr in (left, right):
        pl.semaphore_signal(
            barrier, inc=1,
            device_id=(my_x, nbr, my_z),
            device_id_type=pl.DeviceIdType.MESH,
        )
    pl.semaphore_wait(barrier, 2)

    xg_ref[pl.ds(my_y * TPD, TPD), :] = x_ref[...]
    rg_ref[pl.ds(my_y * EPD, EPD), :] = rt_ref[...]

    for h in range(N_Y - 1):
        o = (my_y - h) % N_Y
        x_rdma = pltpu.make_async_remote_copy(
            src_ref=xg_ref.at[pl.ds(o * TPD, TPD), :],
            dst_ref=xg_ref.at[pl.ds(o * TPD, TPD), :],
            send_sem=ag_ss.at[h],
            recv_sem=ag_rs.at[h],
            device_id=(my_x, right, my_z),
            device_id_type=pl.DeviceIdType.MESH,
        )
        r_rdma = pltpu.make_async_remote_copy(
            src_ref=rg_ref.at[pl.ds(o * EPD, EPD), :],
            dst_ref=rg_ref.at[pl.ds(o * EPD, EPD), :],
            send_sem=rg_ss.at[h],
            recv_sem=rg_rs.at[h],
            device_id=(my_x, right, my_z),
            device_id_type=pl.DeviceIdType.MESH,
        )
        x_rdma.start()
        r_rdma.start()
        x_rdma.wait()
        r_rdma.wait()

    xg = xg_ref[...]
    gates = lax.dot_general(
        xg, rg_ref[...], (((1,), (1,)), ((), ())),
        preferred_element_type=jnp.float32,
        precision=lax.Precision.HIGHEST,
    )

    iota_e = lax.broadcasted_iota(jnp.int32, (T, E), 1)
    m1 = jnp.max(gates, axis=1, keepdims=True)
    i1 = jnp.min(jnp.where(gates == m1, iota_e, E), axis=1, keepdims=True)
    masked = jnp.where(iota_e == i1, -jnp.inf, gates)
    m2 = jnp.max(masked, axis=1, keepdims=True)
    i2 = jnp.min(jnp.where(masked == m2, iota_e, E), axis=1, keepdims=True)
    r = jnp.exp(m2 - m1)
    w_top1 = 1.0 / (1.0 + r)
    w_top2 = r / (1.0 + r)

    xb = xg.astype(jnp.bfloat16)
    acc = jnp.zeros((T, D), jnp.float32)
    for j in range(EPD):
        e_id = my_y * EPD + j
        h1 = jnp.dot(xb, w1_ref[j].astype(jnp.bfloat16),
                     preferred_element_type=jnp.float32)
        h1 = jnp.maximum(h1, 0.0).astype(jnp.bfloat16)
        yj = jnp.dot(h1, w2_ref[j].astype(jnp.bfloat16),
                     preferred_element_type=jnp.float32)
        wt = (w_top1 * (i1 == e_id).astype(jnp.float32)
              + w_top2 * (i2 == e_id).astype(jnp.float32))
        acc = acc + yj * wt
    contrib_ref[...] = acc

    for s in range(N_Y - 1):
        b_send = (my_y - 1 - s) % N_Y
        if s == 0:
            src = contrib_ref.at[pl.ds(b_send * TPD, TPD), :]
        else:
            src = rs_ref.at[s - 1]
        rdma = pltpu.make_async_remote_copy(
            src_ref=src,
            dst_ref=rs_ref.at[s],
            send_sem=rs_ss.at[s],
            recv_sem=rs_rs.at[s],
            device_id=(my_x, right, my_z),
            device_id_type=pl.DeviceIdType.MESH,
        )
        rdma.start()
        rdma.wait()
        b_recv = (my_y - 2 - s) % N_Y
        if s < N_Y - 2:
            rs_ref[s, :, :] = (rs_ref[s, :, :]
                               + contrib_ref[pl.ds(b_recv * TPD, TPD), :])
    out_ref[...] = (rs_ref[N_Y - 2, :, :]
                    + contrib_ref[pl.ds(my_y * TPD, TPD), :])


def kernel(x, router, W1, W2):
    rt = router.T
    return pl.pallas_call(
        _body,
        out_shape=jax.ShapeDtypeStruct((TPD, D), jnp.float32),
        in_specs=[pl.BlockSpec(memory_space=pltpu.VMEM)] * 4,
        out_specs=pl.BlockSpec(memory_space=pltpu.VMEM),
        scratch_shapes=[
            pltpu.VMEM((T, D), jnp.float32),
            pltpu.VMEM((E, D), jnp.float32),
            pltpu.VMEM((T, D), jnp.float32),
            pltpu.VMEM((N_Y - 1, TPD, D), jnp.float32),
            pltpu.SemaphoreType.DMA((N_Y - 1,)),
            pltpu.SemaphoreType.DMA((N_Y - 1,)),
            pltpu.SemaphoreType.DMA((N_Y - 1,)),
            pltpu.SemaphoreType.DMA((N_Y - 1,)),
            pltpu.SemaphoreType.DMA((N_Y - 1,)),
            pltpu.SemaphoreType.DMA((N_Y - 1,)),
        ],
        compiler_params=pltpu.CompilerParams(collective_id=0),
    )(x, rt, W1, W2)


# baseline (device time: 45952 ns/iter reference)
import jax
import jax.numpy as jnp
from jax import lax
from jax.experimental import pallas as pl
from jax.experimental.pallas import tpu as pltpu

N_Y = 4
T = 512
TPD = T // N_Y
D = 512
E = 8
EPD = E // N_Y
F = 1024

NEG_INF = float("-inf")


def _body(x_ref, rt_ref, w1_ref, w2_ref, out_ref,
          xg_ref, rg_ref, contrib_ref, rs_ref,
          ag_ss, ag_rs, rg_ss, rg_rs, rs_ss, rs_rs):
    my_x = lax.axis_index("x")
    my_y = lax.axis_index("y")
    my_z = lax.axis_index("z")
    left = (my_y - 1) % N_Y
    right = (my_y + 1) % N_Y

    barrier = pltpu.get_barrier_semaphore()
    for nbr in (left, right):
        pl.semaphore_signal(
            barrier, inc=1,
            device_id=(my_x, nbr, my_z),
            device_id_type=pl.DeviceIdType.MESH,
        )
    pl.semaphore_wait(barrier, 2)

    xg_ref[my_y] = x_ref[...]
    rg_ref[my_y] = rt_ref[...]

    for h in range(N_Y - 1):
        o = (my_y - h) % N_Y
        x_rdma = pltpu.make_async_remote_copy(
            src_ref=xg_ref.at[o],
            dst_ref=xg_ref.at[o],
            send_sem=ag_ss.at[h],
            recv_sem=ag_rs.at[h],
            device_id=(my_x, right, my_z),
            device_id_type=pl.DeviceIdType.MESH,
        )
        r_rdma = pltpu.make_async_remote_copy(
            src_ref=rg_ref.at[o],
            dst_ref=rg_ref.at[o],
            send_sem=rg_ss.at[h],
            recv_sem=rg_rs.at[h],
            device_id=(my_x, right, my_z),
            device_id_type=pl.DeviceIdType.MESH,
        )
        x_rdma.start()
        r_rdma.start()
        x_rdma.wait()
        r_rdma.wait()

    for b in range(N_Y):
        xb_f32 = xg_ref[b]

        m1 = jnp.full((TPD, 1), NEG_INF, jnp.float32)
        m2 = jnp.full((TPD, 1), NEG_INF, jnp.float32)
        i1 = jnp.zeros((TPD, 1), jnp.int32)
        i2 = jnp.zeros((TPD, 1), jnp.int32)
        for o in range(N_Y):
            g_o = lax.dot_general(
                xb_f32, rg_ref[o], (((1,), (1,)), ((), ())),
                preferred_element_type=jnp.float32,
                precision=lax.Precision.HIGHEST,
            )
            for j in range(EPD):
                e_id = o * EPD + j
                g = g_o[:, j:j + 1]
                beats1 = g > m1
                beats2 = g > m2
                m2 = jnp.where(beats1, m1, jnp.where(beats2, g, m2))
                i2 = jnp.where(beats1, i1, jnp.where(beats2, e_id, i2))
                m1 = jnp.where(beats1, g, m1)
                i1 = jnp.where(beats1, e_id, i1)
        r = jnp.exp(m2 - m1)
        w_top1 = 1.0 / (1.0 + r)
        w_top2 = r / (1.0 + r)

        xb = xb_f32.astype(jnp.bfloat16)
        acc = jnp.zeros((TPD, D), jnp.float32)
        for j in range(EPD):
            e_id = my_y * EPD + j
            h1 = jnp.dot(xb, w1_ref[j].astype(jnp.bfloat16),
                         preferred_element_type=jnp.float32)
            h1 = jnp.maximum(h1, 0.0).astype(jnp.bfloat16)
            yj = jnp.dot(h1, w2_ref[j].astype(jnp.bfloat16),
                         preferred_element_type=jnp.float32)
            wt = (w_top1 * (i1 == e_id).astype(jnp.float32)
                  + w_top2 * (i2 == e_id).astype(jnp.float32))
            acc = acc + yj * wt
        contrib_ref[b] = acc

    for s in range(N_Y - 1):
        b_send = (my_y - 1 - s) % N_Y
        if s == 0:
            src = contrib_ref.at[b_send]
        else:
            src = rs_ref.at[s - 1]
        rdma = pltpu.make_async_remote_copy(
            src_ref=src,
            dst_ref=rs_ref.at[s],
            send_sem=rs_ss.at[s],
            recv_sem=rs_rs.at[s],
            device_id=(my_x, right, my_z),
            device_id_type=pl.DeviceIdType.MESH,
        )
        rdma.start()
        rdma.wait()
        b_recv = (my_y - 2 - s) % N_Y
        if s < N_Y - 2:
            rs_ref[s] = rs_ref[s] + contrib_ref[b_recv]
    out_ref[...] = rs_ref[N_Y - 2] + contrib_ref[my_y]


def kernel(x, router, W1, W2):
    rt = router.T
    return pl.pallas_call(
        _body,
        out_shape=jax.ShapeDtypeStruct((TPD, D), jnp.float32),
        in_specs=[pl.BlockSpec(memory_space=pltpu.VMEM)] * 4,
        out_specs=pl.BlockSpec(memory_space=pltpu.VMEM),
        scratch_shapes=[
            pltpu.VMEM((N_Y, TPD, D), jnp.float32),
            pltpu.VMEM((N_Y, EPD, D), jnp.float32),
            pltpu.VMEM((N_Y, TPD, D), jnp.float32),
            pltpu.VMEM((N_Y - 1, TPD, D), jnp.float32),
            pltpu.SemaphoreType.DMA((N_Y - 1,)),
            pltpu.SemaphoreType.DMA((N_Y - 1,)),
            pltpu.SemaphoreType.DMA((N_Y - 1,)),
            pltpu.SemaphoreType.DMA((N_Y - 1,)),
            pltpu.SemaphoreType.DMA((N_Y - 1,)),
            pltpu.SemaphoreType.DMA((N_Y - 1,)),
        ],
        compiler_params=pltpu.CompilerParams(collective_id=0),
    )(x, rt, W1, W2)


# device time: 27330 ns/iter; 1.6814x vs baseline; 1.6814x over previous
import jax
import jax.numpy as jnp
from jax import lax
from jax.experimental import pallas as pl
from jax.experimental.pallas import tpu as pltpu

N_Y = 4
T = 512
TPD = T // N_Y
D = 512
E = 8
EPD = E // N_Y
F = 1024

NEG_INF = float("-inf")
_MESH = pl.DeviceIdType.MESH


def _body(x_ref, rt_ref, w1_ref, w2_ref, out_ref,
          xg_ref, rg_ref, pay_ref, cbuf_ref, rbuf_ref,
          x_ss, x_rs, rt_ss, rt_rs, py_ss, py_rs, c_ss, c_rs):
    my_x = lax.axis_index("x")
    my_y = lax.axis_index("y")
    my_z = lax.axis_index("z")

    barrier = pltpu.get_barrier_semaphore()
    for k in range(N_Y - 1):
        peer = (my_y + 1 + k) % N_Y
        pl.semaphore_signal(barrier, inc=1,
                            device_id=(my_x, peer, my_z),
                            device_id_type=_MESH)
    pl.semaphore_wait(barrier, N_Y - 1)

    rg_ref[my_y] = rt_ref[...]
    xg_ref[my_y] = x_ref[...].astype(jnp.bfloat16)

    sends = []

    for k in range(N_Y - 1):
        peer = (my_y + 1 + k) % N_Y
        r = pltpu.make_async_remote_copy(
            src_ref=rg_ref.at[my_y], dst_ref=rg_ref.at[my_y],
            send_sem=rt_ss.at[k], recv_sem=rt_rs.at[my_y],
            device_id=(my_x, peer, my_z), device_id_type=_MESH)
        r.start()
        sends.append(r)

    for k in range(N_Y - 1):
        peer = (my_y + 1 + k) % N_Y
        r = pltpu.make_async_remote_copy(
            src_ref=xg_ref.at[my_y], dst_ref=xg_ref.at[my_y],
            send_sem=x_ss.at[k], recv_sem=x_rs.at[my_y],
            device_id=(my_x, peer, my_z), device_id_type=_MESH)
        r.start()
        sends.append(r)

    def _wait_recv(buf_ref, sem_arr, slot):
        pltpu.make_async_remote_copy(
            src_ref=buf_ref.at[slot], dst_ref=buf_ref.at[slot],
            send_sem=sem_arr.at[slot], recv_sem=sem_arr.at[slot],
            device_id=(my_x, my_y, my_z), device_id_type=_MESH,
        ).wait_recv()

    for k in range(N_Y - 1):
        peer = (my_y + 1 + k) % N_Y
        _wait_recv(rg_ref, rt_rs, peer)

    xf = x_ref[...]
    m1 = jnp.full((TPD, 1), NEG_INF, jnp.float32)
    m2 = jnp.full((TPD, 1), NEG_INF, jnp.float32)
    i1 = jnp.zeros((TPD, 1), jnp.int32)
    i2 = jnp.zeros((TPD, 1), jnp.int32)
    for o in range(N_Y):
        g_o = lax.dot_general(
            xf, rg_ref[o], (((1,), (1,)), ((), ())),
            preferred_element_type=jnp.float32,
            precision=lax.Precision.HIGHEST,
        )
        for j in range(EPD):
            e_id = o * EPD + j
            g = g_o[:, j:j + 1]
            beats1 = g > m1
            beats2 = g > m2
            m2 = jnp.where(beats1, m1, jnp.where(beats2, g, m2))
            i2 = jnp.where(beats1, i1, jnp.where(beats2, e_id, i2))
            m1 = jnp.where(beats1, g, m1)
            i1 = jnp.where(beats1, e_id, i1)
    r = jnp.exp(m2 - m1)
    w_top1 = 1.0 / (1.0 + r)
    w_top2 = r / (1.0 + r)

    lanes = lax.broadcasted_iota(jnp.int32, (TPD, E), 1)
    pay = (jnp.where(lanes == 0, i1.astype(jnp.float32), 0.0)
           + jnp.where(lanes == 1, i2.astype(jnp.float32), 0.0)
           + jnp.where(lanes == 2, w_top1, 0.0)
           + jnp.where(lanes == 3, w_top2, 0.0))
    pay_ref[my_y] = pay
    for k in range(N_Y - 1):
        peer = (my_y + 1 + k) % N_Y
        r = pltpu.make_async_remote_copy(
            src_ref=pay_ref.at[my_y], dst_ref=pay_ref.at[my_y],
            send_sem=py_ss.at[k], recv_sem=py_rs.at[my_y],
            device_id=(my_x, peer, my_z), device_id_type=_MESH)
        r.start()
        sends.append(r)

    w1b = [w1_ref[j].astype(jnp.bfloat16) for j in range(EPD)]
    w2b = [w2_ref[j].astype(jnp.bfloat16) for j in range(EPD)]
    own_acc = None
    for s in range(N_Y):
        b = (my_y + s) % N_Y
        if s > 0:
            _wait_recv(xg_ref, x_rs, b)
        xb = xg_ref[b]
        ys = []
        for j in range(EPD):
            h1 = jnp.dot(xb, w1b[j], preferred_element_type=jnp.float32)
            h1 = jnp.maximum(h1, 0.0).astype(jnp.bfloat16)
            ys.append(jnp.dot(h1, w2b[j],
                              preferred_element_type=jnp.float32))
        if s > 0:
            _wait_recv(pay_ref, py_rs, b)
        pay_b = pay_ref[b]
        i1v = pay_b[:, 0:1].astype(jnp.int32)
        i2v = pay_b[:, 1:2].astype(jnp.int32)
        w1v = pay_b[:, 2:3]
        w2v = pay_b[:, 3:4]
        acc = jnp.zeros((TPD, D), jnp.float32)
        for j in range(EPD):
            e_id = my_y * EPD + j
            wt = (w1v * (i1v == e_id).astype(jnp.float32)
                  + w2v * (i2v == e_id).astype(jnp.float32))
            acc = acc + ys[j] * wt
        if s == 0:
            own_acc = acc
        else:
            cbuf_ref[b] = acc.astype(jnp.bfloat16)
            r = pltpu.make_async_remote_copy(
                src_ref=cbuf_ref.at[b], dst_ref=rbuf_ref.at[my_y],
                send_sem=c_ss.at[s - 1], recv_sem=c_rs.at[my_y],
                device_id=(my_x, b, my_z), device_id_type=_MESH)
            r.start()
            sends.append(r)

    total = own_acc
    for k in range(N_Y - 1):
        peer = (my_y + 1 + k) % N_Y
        _wait_recv(rbuf_ref, c_rs, peer)
        total = total + rbuf_ref[peer].astype(jnp.float32)
    out_ref[...] = total

    for snd in sends:
        snd.wait_send()


def kernel(x, router, W1, W2):
    rt = router.T
    return pl.pallas_call(
        _body,
        out_shape=jax.ShapeDtypeStruct((TPD, D), jnp.float32),
        in_specs=[pl.BlockSpec(memory_space=pltpu.VMEM)] * 4,
        out_specs=pl.BlockSpec(memory_space=pltpu.VMEM),
        scratch_shapes=[
            pltpu.VMEM((N_Y, TPD, D), jnp.bfloat16),
            pltpu.VMEM((N_Y, EPD, D), jnp.float32),
            pltpu.VMEM((N_Y, TPD, E), jnp.float32),
            pltpu.VMEM((N_Y, TPD, D), jnp.bfloat16),
            pltpu.VMEM((N_Y, TPD, D), jnp.bfloat16),
            pltpu.SemaphoreType.DMA((N_Y - 1,)),
            pltpu.SemaphoreType.DMA((N_Y,)),
            pltpu.SemaphoreType.DMA((N_Y - 1,)),
            pltpu.SemaphoreType.DMA((N_Y,)),
            pltpu.SemaphoreType.DMA((N_Y - 1,)),
            pltpu.SemaphoreType.DMA((N_Y,)),
            pltpu.SemaphoreType.DMA((N_Y - 1,)),
            pltpu.SemaphoreType.DMA((N_Y,)),
        ],
        compiler_params=pltpu.CompilerParams(collective_id=0),
    )(x, rt, W1, W2)


# device time: 27312 ns/iter; 1.6825x vs baseline; 1.0007x over previous
import jax
import jax.numpy as jnp
from jax import lax
from jax.experimental import pallas as pl
from jax.experimental.pallas import tpu as pltpu

N_Y = 4
T = 512
TPD = T // N_Y
D = 512
E = 8
EPD = E // N_Y
F = 1024

NEG_INF = float("-inf")
_MESH = pl.DeviceIdType.MESH


def _body(x_ref, rt_ref, w1_ref, w2_ref, out_ref,
          xg_ref, rg_ref, pay_ref, cbuf_ref, rbuf_ref,
          x_ss, x_rs, rt_ss, rt_rs, py_ss, py_rs, c_ss, c_rs):
    my_x = lax.axis_index("x")
    my_y = lax.axis_index("y")
    my_z = lax.axis_index("z")

    barrier = pltpu.get_barrier_semaphore()
    for k in range(N_Y - 1):
        peer = (my_y + 1 + k) % N_Y
        pl.semaphore_signal(barrier, inc=1,
                            device_id=(my_x, peer, my_z),
                            device_id_type=_MESH)
    pl.semaphore_wait(barrier, N_Y - 1)

    rg_ref[my_y] = rt_ref[...]
    xg_ref[my_y] = x_ref[...].astype(jnp.bfloat16)

    sends = []

    for k in range(N_Y - 1):
        peer = (my_y + 1 + k) % N_Y
        r = pltpu.make_async_remote_copy(
            src_ref=rg_ref.at[my_y], dst_ref=rg_ref.at[my_y],
            send_sem=rt_ss.at[k], recv_sem=rt_rs.at[my_y],
            device_id=(my_x, peer, my_z), device_id_type=_MESH)
        r.start()
        sends.append(r)

    for k in range(N_Y - 1):
        peer = (my_y + 1 + k) % N_Y
        r = pltpu.make_async_remote_copy(
            src_ref=xg_ref.at[my_y], dst_ref=xg_ref.at[my_y],
            send_sem=x_ss.at[k], recv_sem=x_rs.at[my_y],
            device_id=(my_x, peer, my_z), device_id_type=_MESH)
        r.start()
        sends.append(r)

    def _wait_recv(buf_ref, sem_arr, slot):
        pltpu.make_async_remote_copy(
            src_ref=buf_ref.at[slot], dst_ref=buf_ref.at[slot],
            send_sem=sem_arr.at[slot], recv_sem=sem_arr.at[slot],
            device_id=(my_x, my_y, my_z), device_id_type=_MESH,
        ).wait_recv()

    for k in range(N_Y - 1):
        peer = (my_y + 1 + k) % N_Y
        _wait_recv(rg_ref, rt_rs, peer)

    xf = x_ref[...]
    m1 = jnp.full((TPD, 1), NEG_INF, jnp.float32)
    m2 = jnp.full((TPD, 1), NEG_INF, jnp.float32)
    i1 = jnp.zeros((TPD, 1), jnp.int32)
    i2 = jnp.zeros((TPD, 1), jnp.int32)
    for o in range(N_Y):
        g_o = lax.dot_general(
            xf, rg_ref[o], (((1,), (1,)), ((), ())),
            preferred_element_type=jnp.float32,
            precision=lax.Precision.HIGHEST,
        )
        for j in range(EPD):
            e_id = o * EPD + j
            g = g_o[:, j:j + 1]
            beats1 = g > m1
            beats2 = g > m2
            m2 = jnp.where(beats1, m1, jnp.where(beats2, g, m2))
            i2 = jnp.where(beats1, i1, jnp.where(beats2, e_id, i2))
            m1 = jnp.where(beats1, g, m1)
            i1 = jnp.where(beats1, e_id, i1)
    r = jnp.exp(m2 - m1)
    w_top1 = 1.0 / (1.0 + r)
    w_top2 = r / (1.0 + r)

    lanes = lax.broadcasted_iota(jnp.int32, (TPD, E), 1)
    pay = (jnp.where(lanes == 0, i1.astype(jnp.float32), 0.0)
           + jnp.where(lanes == 1, i2.astype(jnp.float32), 0.0)
           + jnp.where(lanes == 2, w_top1, 0.0)
           + jnp.where(lanes == 3, w_top2, 0.0))
    pay_ref[my_y] = pay
    for k in range(N_Y - 1):
        peer = (my_y + 1 + k) % N_Y
        r = pltpu.make_async_remote_copy(
            src_ref=pay_ref.at[my_y], dst_ref=pay_ref.at[my_y],
            send_sem=py_ss.at[k], recv_sem=py_rs.at[my_y],
            device_id=(my_x, peer, my_z), device_id_type=_MESH)
        r.start()
        sends.append(r)

    own_acc = None
    for s in range(N_Y):
        b = (my_y + s) % N_Y
        if s > 0:
            _wait_recv(xg_ref, x_rs, b)
        xb = xg_ref[b]
        ys = []
        for j in range(EPD):
            h1 = jnp.dot(xb.astype(jnp.float32), w1_ref[j],
                         preferred_element_type=jnp.float32)
            h1 = jnp.maximum(h1, 0.0)
            ys.append(jnp.dot(h1, w2_ref[j],
                              preferred_element_type=jnp.float32))
        if s > 0:
            _wait_recv(pay_ref, py_rs, b)
        pay_b = pay_ref[b]
        i1v = pay_b[:, 0:1].astype(jnp.int32)
        i2v = pay_b[:, 1:2].astype(jnp.int32)
        w1v = pay_b[:, 2:3]
        w2v = pay_b[:, 3:4]
        acc = jnp.zeros((TPD, D), jnp.float32)
        for j in range(EPD):
            e_id = my_y * EPD + j
            wt = (w1v * (i1v == e_id).astype(jnp.float32)
                  + w2v * (i2v == e_id).astype(jnp.float32))
            acc = acc + ys[j] * wt
        if s == 0:
            own_acc = acc
        else:
            cbuf_ref[b] = acc.astype(jnp.bfloat16)
            r = pltpu.make_async_remote_copy(
                src_ref=cbuf_ref.at[b], dst_ref=rbuf_ref.at[my_y],
                send_sem=c_ss.at[s - 1], recv_sem=c_rs.at[my_y],
                device_id=(my_x, b, my_z), device_id_type=_MESH)
            r.start()
            sends.append(r)

    total = own_acc
    for k in range(N_Y - 1):
        peer = (my_y + 1 + k) % N_Y
        _wait_recv(rbuf_ref, c_rs, peer)
        total = total + rbuf_ref[peer].astype(jnp.float32)
    out_ref[...] = total

    for snd in sends:
        snd.wait_send()


def kernel(x, router, W1, W2):
    rt = router.T
    return pl.pallas_call(
        _body,
        out_shape=jax.ShapeDtypeStruct((TPD, D), jnp.float32),
        in_specs=[pl.BlockSpec(memory_space=pltpu.VMEM)] * 4,
        out_specs=pl.BlockSpec(memory_space=pltpu.VMEM),
        scratch_shapes=[
            pltpu.VMEM((N_Y, TPD, D), jnp.bfloat16),
            pltpu.VMEM((N_Y, EPD, D), jnp.float32),
            pltpu.VMEM((N_Y, TPD, E), jnp.float32),
            pltpu.VMEM((N_Y, TPD, D), jnp.bfloat16),
            pltpu.VMEM((N_Y, TPD, D), jnp.bfloat16),
            pltpu.SemaphoreType.DMA((N_Y - 1,)),
            pltpu.SemaphoreType.DMA((N_Y,)),
            pltpu.SemaphoreType.DMA((N_Y - 1,)),
            pltpu.SemaphoreType.DMA((N_Y,)),
            pltpu.SemaphoreType.DMA((N_Y - 1,)),
            pltpu.SemaphoreType.DMA((N_Y,)),
            pltpu.SemaphoreType.DMA((N_Y - 1,)),
            pltpu.SemaphoreType.DMA((N_Y,)),
        ],
        compiler_params=pltpu.CompilerParams(collective_id=0),
    )(x, rt, W1, W2)
